# baseline (speedup 1.0000x reference)
import jax
import jax.numpy as jnp
from jax.experimental import pallas as pl
from jax.experimental.pallas import tpu as pltpu

_TILE = 1024


def _gating_kernel(x_ref, w_ref, b_ref, out_ref):
    logits = jax.lax.dot_general(
        x_ref[...], w_ref[...],
        dimension_numbers=(((1,), (1,)), ((), ())),
        preferred_element_type=jnp.float32,
    )
    logits = logits + b_ref[...]
    m = jnp.max(logits, axis=-1, keepdims=True)
    e = jnp.exp(logits - m)
    s = jnp.sum(e, axis=-1, keepdims=True)
    out_ref[...] = e / s


def kernel(x, W, b):
    n_tokens, input_dim = x.shape
    num_experts = W.shape[0]
    b2 = b.reshape(1, num_experts)
    return pl.pallas_call(
        _gating_kernel,
        grid=(n_tokens // _TILE,),
        in_specs=[
            pl.BlockSpec((_TILE, input_dim), lambda i: (i, 0)),
            pl.BlockSpec((num_experts, input_dim), lambda i: (0, 0)),
            pl.BlockSpec((1, num_experts), lambda i: (0, 0)),
        ],
        out_specs=pl.BlockSpec((_TILE, num_experts), lambda i: (i, 0)),
        out_shape=jax.ShapeDtypeStruct((n_tokens, num_experts), jnp.float32),
        compiler_params=pltpu.CompilerParams(
            dimension_semantics=("arbitrary",),
        ),
    )(x, W, b2)
